# baseline (speedup 1.0000x reference)
import functools

import jax
import jax.numpy as jnp
from jax import lax
from jax.experimental import pallas as pl
from jax.experimental.pallas import tpu as pltpu
from jax.experimental.pallas import tpu_sc as plsc

N = 10000
E = 320000
MEM = 128
INJ = 0.1

NC = 2
NS = 16
NW = NC * NS
EPT = E // NW
CH = 40
NCHUNK = EPT // CH
RZB = 624
RTAIL = N - NS * RZB

_f32 = jnp.float32


def _node_proj_body(x_ref, win_ref, bin_ref, a_ref, b_ref,
                    mem_ref, ps_ref, pd_ref):
    x = x_ref[...]
    m = jnp.tanh(jnp.dot(x, win_ref[...], preferred_element_type=_f32)
                 + bin_ref[...])
    mem_ref[...] = m
    ps_ref[...] = jnp.dot(m, a_ref[...], preferred_element_type=_f32)
    pd_ref[...] = jnp.dot(m, b_ref[...], preferred_element_type=_f32)


def _edge_term_body(z_ref, c_ref, bm1_ref, ez_ref):
    ez_ref[...] = (jnp.dot(z_ref[...], c_ref[...], preferred_element_type=_f32)
                   + bm1_ref[...])


def _finalize_body(p0_ref, p1_ref, c0_ref, c1_ref, mem_ref,
                   wm2_ref, bm2_ref, wmm_ref, bmm_ref,
                   wzt_ref, wzb_ref, bz_ref,
                   wrt_ref, wrb_ref, br_ref,
                   wht_ref, whb_ref, bh_ref,
                   wout_ref, bout_ref, out_ref):
    s = p0_ref[...] + p1_ref[...]
    cnt = c0_ref[...][:, 0:1] + c1_ref[...][:, 0:1]
    agg = (jnp.dot(s, wm2_ref[...], preferred_element_type=_f32)
           + cnt * bm2_ref[...]) / jnp.maximum(cnt, 1.0)
    am = jnp.dot(agg, wmm_ref[...], preferred_element_type=_f32) + bmm_ref[...]
    m = mem_ref[...]
    z = jax.nn.sigmoid(jnp.dot(am, wzt_ref[...], preferred_element_type=_f32)
                       + jnp.dot(m, wzb_ref[...], preferred_element_type=_f32)
                       + bz_ref[...])
    r = jax.nn.sigmoid(jnp.dot(am, wrt_ref[...], preferred_element_type=_f32)
                       + jnp.dot(m, wrb_ref[...], preferred_element_type=_f32)
                       + br_ref[...])
    ht = jnp.tanh(jnp.dot(am, wht_ref[...], preferred_element_type=_f32)
                  + jnp.dot(r * m, whb_ref[...], preferred_element_type=_f32)
                  + bh_ref[...])
    mn = (1.0 - z) * m + z * ht + INJ * m
    out_ref[...] = jnp.maximum(
        jnp.dot(mn, wout_ref[...], preferred_element_type=_f32)
        + bout_ref[...], 0.0)


def _rep(shape):
    return pl.BlockSpec(shape, lambda i: (0, 0))


_NODE_BLK = 1000
_node_proj = pl.pallas_call(
    _node_proj_body,
    grid=(N // _NODE_BLK,),
    in_specs=[
        pl.BlockSpec((_NODE_BLK, MEM), lambda i: (i, 0)),
        _rep((MEM, MEM)), _rep((1, MEM)), _rep((MEM, MEM)), _rep((MEM, MEM)),
    ],
    out_specs=[pl.BlockSpec((_NODE_BLK, MEM), lambda i: (i, 0))] * 3,
    out_shape=[jax.ShapeDtypeStruct((N, MEM), _f32)] * 3,
)

_EDGE_BLK = 4000
_edge_term = pl.pallas_call(
    _edge_term_body,
    grid=(E // _EDGE_BLK,),
    in_specs=[
        pl.BlockSpec((_EDGE_BLK, 16), lambda i: (i, 0)),
        _rep((16, MEM)), _rep((1, MEM)),
    ],
    out_specs=pl.BlockSpec((_EDGE_BLK, MEM), lambda i: (i, 0)),
    out_shape=jax.ShapeDtypeStruct((E, MEM), _f32),
)

_finalize = pl.pallas_call(
    _finalize_body,
    grid=(N // _NODE_BLK,),
    in_specs=[
        pl.BlockSpec((_NODE_BLK, MEM), lambda i: (i, 0)),
        pl.BlockSpec((_NODE_BLK, MEM), lambda i: (i, 0)),
        pl.BlockSpec((_NODE_BLK, MEM), lambda i: (i, 0)),
        pl.BlockSpec((_NODE_BLK, MEM), lambda i: (i, 0)),
        pl.BlockSpec((_NODE_BLK, MEM), lambda i: (i, 0)),
        _rep((MEM, MEM)), _rep((1, MEM)),
        _rep((MEM, MEM)), _rep((1, MEM)),
        _rep((MEM, MEM)), _rep((MEM, MEM)), _rep((1, MEM)),
        _rep((MEM, MEM)), _rep((MEM, MEM)), _rep((1, MEM)),
        _rep((MEM, MEM)), _rep((MEM, MEM)), _rep((1, MEM)),
        _rep((MEM, MEM)), _rep((1, MEM)),
    ],
    out_specs=pl.BlockSpec((_NODE_BLK, MEM), lambda i: (i, 0)),
    out_shape=jax.ShapeDtypeStruct((N, MEM), _f32),
)


def _edge_agg_body(ps_hbm, pd_hbm, ez_hbm, src_hbm, dst_hbm,
                   z128_hbm, ones_hbm,
                   out_hbm, cnt_hbm,
                   acc_sh, srcv0, srcv1, dstv0, dstv1, dsts0, dsts1,
                   rowsa0, rowsa1, rowsb0, rowsb1, rowse0, rowse1,
                   sem_a0, sem_a1, sem_b0, sem_b1, sem_e0, sem_e1,
                   sem_i0, sem_i1, sem_s0, sem_s1):
    cid = lax.axis_index("c")
    sid = lax.axis_index("s")
    wid = cid * NS + sid
    r0 = sid * RZB
    t0 = NS * RZB
    e_base = wid * EPT
    srcv = (srcv0, srcv1)
    dstv = (dstv0, dstv1)
    dsts = (dsts0, dsts1)
    rowsa = (rowsa0, rowsa1)
    rowsb = (rowsb0, rowsb1)
    rowse = (rowse0, rowse1)
    sem_a = (sem_a0, sem_a1)
    sem_b = (sem_b0, sem_b1)
    sem_e = (sem_e0, sem_e1)
    sem_i = (sem_i0, sem_i1)
    sem_s = (sem_s0, sem_s1)

    def zero_acc():
        pltpu.sync_copy(z128_hbm.at[pl.ds(r0, RZB)],
                        acc_sh.at[pl.ds(r0, RZB)])

        @pl.when(sid == NS - 1)
        def _():
            pltpu.sync_copy(z128_hbm.at[pl.ds(t0, RTAIL)],
                            acc_sh.at[pl.ds(t0, RTAIL)])

    def write_acc(dst3d):
        pltpu.sync_copy(acc_sh.at[pl.ds(r0, RZB)],
                        dst3d.at[cid, pl.ds(r0, RZB)])

        @pl.when(sid == NS - 1)
        def _():
            pltpu.sync_copy(acc_sh.at[pl.ds(t0, RTAIL)],
                            dst3d.at[cid, pl.ds(t0, RTAIL)])

    def idx_copy_async(t, p):
        e0 = e_base + t * CH
        pltpu.async_copy(src_hbm.at[pl.ds(e0, CH)], srcv[p], sem_i[p])
        return pltpu.async_copy(dst_hbm.at[pl.ds(e0, CH)], dstv[p], sem_i[p])

    def gathers_async(t, p):
        e0 = e_base + t * CH
        pltpu.async_copy(ps_hbm.at[srcv[p]], rowsa[p], sem_a[p])
        pltpu.async_copy(pd_hbm.at[dstv[p]], rowsb[p], sem_b[p])
        return pltpu.async_copy(ez_hbm.at[pl.ds(e0, CH)], rowse[p], sem_e[p])

    zero_acc()
    plsc.subcore_barrier()

    idx_copy_async(0, 0)
    pltpu.make_async_copy(src_hbm.at[pl.ds(0, CH)], srcv[0], sem_i[0]).wait()
    pltpu.make_async_copy(dst_hbm.at[pl.ds(0, CH)], dstv[0], sem_i[0]).wait()
    idx_copy_async(1, 1)
    gathers_async(0, 0)

    def chunk(i, carry):
        for b in (0, 1):
            t = 2 * i + b

            pltpu.make_async_copy(ez_hbm.at[pl.ds(0, CH)], rowsa[b],
                                  sem_a[b]).wait()
            pltpu.make_async_copy(ez_hbm.at[pl.ds(0, CH)], rowsb[b],
                                  sem_b[b]).wait()
            pltpu.make_async_copy(ez_hbm.at[pl.ds(0, CH)], rowse[b],
                                  sem_e[b]).wait()

            @pl.when((t >= 1) & (t + 1 < NCHUNK))
            def _():
                pltpu.make_async_copy(rowse[1 - b],
                                      acc_sh.at[dsts[1 - b]],
                                      sem_s[1 - b]).wait()

            @pl.when(t + 1 < NCHUNK)
            def _():
                pltpu.make_async_copy(src_hbm.at[pl.ds(0, CH)],
                                      srcv[1 - b], sem_i[1 - b]).wait()
                pltpu.make_async_copy(dst_hbm.at[pl.ds(0, CH)],
                                      dstv[1 - b], sem_i[1 - b]).wait()
                gathers_async(t + 1, 1 - b)

            @plsc.parallel_loop(0, CH, unroll=2)
            def row(rr):
                for cc in range(MEM // 16):
                    sl = pl.ds(cc * 16, 16)
                    v = rowsa[b][rr, sl] + rowsb[b][rr, sl] + rowse[b][rr, sl]
                    rowse[b][rr, sl] = jnp.maximum(v, 0.0)
            dsts[b][pl.ds(0, 16)] = dstv[b][pl.ds(0, 16)]
            dsts[b][pl.ds(16, 16)] = dstv[b][pl.ds(16, 16)]
            dsts[b][pl.ds(CH - 16, 16)] = dstv[b][pl.ds(CH - 16, 16)]
            pltpu.async_copy(rowse[b], acc_sh.at[dsts[b]], sem_s[b],
                             add=True)

            @pl.when(t + 2 < NCHUNK)
            def _():
                idx_copy_async(t + 2, b)

        return carry

    lax.fori_loop(0, NCHUNK // 2, chunk, 0)
    pltpu.make_async_copy(rowse[0], acc_sh.at[dsts[0]], sem_s[0]).wait()
    pltpu.make_async_copy(rowse[1], acc_sh.at[dsts[1]], sem_s[1]).wait()
    plsc.subcore_barrier()
    write_acc(out_hbm)
    plsc.subcore_barrier()

    zero_acc()
    pltpu.sync_copy(ones_hbm, rowse[0])
    plsc.subcore_barrier()

    def dst_copy_async(t, p):
        e0 = e_base + t * CH
        return pltpu.async_copy(dst_hbm.at[pl.ds(e0, CH)], dstv[p], sem_i[p])

    dst_copy_async(0, 0).wait()
    dst_copy_async(1, 1)

    def cchunk(i, carry):
        for b in (0, 1):
            t = 2 * i + b

            @pl.when(t >= 2)
            def _():
                pltpu.make_async_copy(rowse[0], acc_sh.at[dsts[b]],
                                      sem_s[b]).wait()

            dsts[b][pl.ds(0, 16)] = dstv[b][pl.ds(0, 16)]
            dsts[b][pl.ds(16, 16)] = dstv[b][pl.ds(16, 16)]
            dsts[b][pl.ds(CH - 16, 16)] = dstv[b][pl.ds(CH - 16, 16)]
            pltpu.async_copy(rowse[0], acc_sh.at[dsts[b]], sem_s[b],
                             add=True)

            @pl.when(t + 2 < NCHUNK)
            def _():
                dst_copy_async(t + 2, b)

            @pl.when(t + 1 < NCHUNK)
            def _():
                pltpu.make_async_copy(dst_hbm.at[pl.ds(0, CH)],
                                      dstv[1 - b], sem_i[1 - b]).wait()

        return carry

    lax.fori_loop(0, NCHUNK // 2, cchunk, 0)
    pltpu.make_async_copy(rowse[0], acc_sh.at[dsts[0]], sem_s[0]).wait()
    pltpu.make_async_copy(rowse[0], acc_sh.at[dsts[1]], sem_s[1]).wait()
    plsc.subcore_barrier()
    write_acc(cnt_hbm)


@functools.lru_cache(maxsize=1)
def _build_edge_agg():
    return functools.partial(
        pl.kernel,
        out_type=[
            jax.ShapeDtypeStruct((NC, N, MEM), _f32),
            jax.ShapeDtypeStruct((NC, N, MEM), _f32),
        ],
        mesh=plsc.VectorSubcoreMesh(
            core_axis_name="c", subcore_axis_name="s",
            num_cores=NC, num_subcores=NS),
        scratch_types=(
            [pltpu.VMEM_SHARED((N, MEM), _f32)]
            + [pltpu.VMEM((CH,), jnp.int32)] * 6
            + [pltpu.VMEM((CH, MEM), _f32)] * 6
            + [pltpu.SemaphoreType.DMA] * 10
        ),
    )(_edge_agg_body)


def kernel(node_x, edge_index, edge_z, W_in, b_in, Wm1, bm1, Wm2, bm2,
           Wmm, bmm, Wz, bz, Wr, br, Wh, bh, Wout, bout):
    src = edge_index[0].astype(jnp.int32)
    dst = edge_index[1].astype(jnp.int32)
    a_w = Wm1[:MEM]
    b_w = Wm1[MEM:2 * MEM]
    c_w = Wm1[2 * MEM:]

    mem, ps, pd = _node_proj(node_x, W_in, b_in.reshape(1, MEM), a_w, b_w)
    ez = _edge_term(edge_z, c_w, bm1.reshape(1, MEM))

    z128 = jnp.zeros((N, MEM), _f32)
    ones = jnp.ones((CH, MEM), _f32)
    partial, cnt = _build_edge_agg()(ps, pd, ez, src, dst, z128, ones)

    emb = _finalize(
        partial[0], partial[1], cnt[0], cnt[1], mem,
        Wm2, bm2.reshape(1, MEM), Wmm, bmm.reshape(1, MEM),
        Wz[:MEM], Wz[MEM:], bz.reshape(1, MEM),
        Wr[:MEM], Wr[MEM:], br.reshape(1, MEM),
        Wh[:MEM], Wh[MEM:], bh.reshape(1, MEM),
        Wout, bout.reshape(1, MEM))
    return emb


# speedup vs baseline: 1.0068x; 1.0068x over previous
import functools

import jax
import jax.numpy as jnp
from jax import lax
from jax.experimental import pallas as pl
from jax.experimental.pallas import tpu as pltpu
from jax.experimental.pallas import tpu_sc as plsc

N = 10000
E = 320000
MEM = 128
INJ = 0.1

NC = 2
NS = 16
NW = NC * NS
EPT = E // NW
CH = 40
NCHUNK = EPT // CH
RZB = 624
RTAIL = N - NS * RZB

_f32 = jnp.float32


def _node_proj_body(x_ref, win_ref, bin_ref, a_ref, b_ref,
                    mem_ref, ps_ref, pd_ref):
    x = x_ref[...]
    m = jnp.tanh(jnp.dot(x, win_ref[...], preferred_element_type=_f32)
                 + bin_ref[...])
    mem_ref[...] = m
    ps_ref[...] = jnp.dot(m, a_ref[...], preferred_element_type=_f32)
    pd_ref[...] = jnp.dot(m, b_ref[...], preferred_element_type=_f32)


def _edge_term_body(z_ref, c_ref, bm1_ref, ez_ref):
    ez_ref[...] = (jnp.dot(z_ref[...], c_ref[...], preferred_element_type=_f32)
                   + bm1_ref[...])


def _finalize_body(p0_ref, p1_ref, c0_ref, c1_ref, mem_ref,
                   wm2_ref, bm2_ref, wmm_ref, bmm_ref,
                   wzt_ref, wzb_ref, bz_ref,
                   wrt_ref, wrb_ref, br_ref,
                   wht_ref, whb_ref, bh_ref,
                   wout_ref, bout_ref, out_ref):
    s = p0_ref[...] + p1_ref[...]
    cnt = c0_ref[...][:, 0:1] + c1_ref[...][:, 0:1]
    agg = (jnp.dot(s, wm2_ref[...], preferred_element_type=_f32)
           + cnt * bm2_ref[...]) / jnp.maximum(cnt, 1.0)
    am = jnp.dot(agg, wmm_ref[...], preferred_element_type=_f32) + bmm_ref[...]
    m = mem_ref[...]
    z = jax.nn.sigmoid(jnp.dot(am, wzt_ref[...], preferred_element_type=_f32)
                       + jnp.dot(m, wzb_ref[...], preferred_element_type=_f32)
                       + bz_ref[...])
    r = jax.nn.sigmoid(jnp.dot(am, wrt_ref[...], preferred_element_type=_f32)
                       + jnp.dot(m, wrb_ref[...], preferred_element_type=_f32)
                       + br_ref[...])
    ht = jnp.tanh(jnp.dot(am, wht_ref[...], preferred_element_type=_f32)
                  + jnp.dot(r * m, whb_ref[...], preferred_element_type=_f32)
                  + bh_ref[...])
    mn = (1.0 - z) * m + z * ht + INJ * m
    out_ref[...] = jnp.maximum(
        jnp.dot(mn, wout_ref[...], preferred_element_type=_f32)
        + bout_ref[...], 0.0)


def _rep(shape):
    return pl.BlockSpec(shape, lambda i: (0, 0))


_NODE_BLK = 1000
_node_proj = pl.pallas_call(
    _node_proj_body,
    grid=(N // _NODE_BLK,),
    in_specs=[
        pl.BlockSpec((_NODE_BLK, MEM), lambda i: (i, 0)),
        _rep((MEM, MEM)), _rep((1, MEM)), _rep((MEM, MEM)), _rep((MEM, MEM)),
    ],
    out_specs=[pl.BlockSpec((_NODE_BLK, MEM), lambda i: (i, 0))] * 3,
    out_shape=[jax.ShapeDtypeStruct((N, MEM), _f32)] * 3,
)

_EDGE_BLK = 4000
_edge_term = pl.pallas_call(
    _edge_term_body,
    grid=(E // _EDGE_BLK,),
    in_specs=[
        pl.BlockSpec((_EDGE_BLK, 16), lambda i: (i, 0)),
        _rep((16, MEM)), _rep((1, MEM)),
    ],
    out_specs=pl.BlockSpec((_EDGE_BLK, MEM), lambda i: (i, 0)),
    out_shape=jax.ShapeDtypeStruct((E, MEM), _f32),
)

_finalize = pl.pallas_call(
    _finalize_body,
    grid=(N // _NODE_BLK,),
    in_specs=[
        pl.BlockSpec((_NODE_BLK, MEM), lambda i: (i, 0)),
        pl.BlockSpec((_NODE_BLK, MEM), lambda i: (i, 0)),
        pl.BlockSpec((_NODE_BLK, MEM), lambda i: (i, 0)),
        pl.BlockSpec((_NODE_BLK, MEM), lambda i: (i, 0)),
        pl.BlockSpec((_NODE_BLK, MEM), lambda i: (i, 0)),
        _rep((MEM, MEM)), _rep((1, MEM)),
        _rep((MEM, MEM)), _rep((1, MEM)),
        _rep((MEM, MEM)), _rep((MEM, MEM)), _rep((1, MEM)),
        _rep((MEM, MEM)), _rep((MEM, MEM)), _rep((1, MEM)),
        _rep((MEM, MEM)), _rep((MEM, MEM)), _rep((1, MEM)),
        _rep((MEM, MEM)), _rep((1, MEM)),
    ],
    out_specs=pl.BlockSpec((_NODE_BLK, MEM), lambda i: (i, 0)),
    out_shape=jax.ShapeDtypeStruct((N, MEM), _f32),
)


def _edge_agg_body(ps_hbm, pd_hbm, ez_hbm, src_hbm, dst_hbm,
                   z128_hbm, ones_hbm,
                   out_hbm, cnt_hbm,
                   acc_sh, srcv0, srcv1, dstv0, dstv1, dsts0, dsts1,
                   rowsa0, rowsa1, rowsb0, rowsb1, rowse0, rowse1,
                   sem_a0, sem_a1, sem_b0, sem_b1, sem_e0, sem_e1,
                   sem_i0, sem_i1, sem_s0, sem_s1):
    cid = lax.axis_index("c")
    sid = lax.axis_index("s")
    wid = cid * NS + sid
    r0 = sid * RZB
    t0 = NS * RZB
    e_base = wid * EPT
    srcv = (srcv0, srcv1)
    dstv = (dstv0, dstv1)
    dsts = (dsts0, dsts1)
    rowsa = (rowsa0, rowsa1)
    rowsb = (rowsb0, rowsb1)
    rowse = (rowse0, rowse1)
    sem_a = (sem_a0, sem_a1)
    sem_b = (sem_b0, sem_b1)
    sem_e = (sem_e0, sem_e1)
    sem_i = (sem_i0, sem_i1)
    sem_s = (sem_s0, sem_s1)

    def zero_acc():
        pltpu.sync_copy(z128_hbm.at[pl.ds(r0, RZB)],
                        acc_sh.at[pl.ds(r0, RZB)])

        @pl.when(sid == NS - 1)
        def _():
            pltpu.sync_copy(z128_hbm.at[pl.ds(t0, RTAIL)],
                            acc_sh.at[pl.ds(t0, RTAIL)])

    def write_acc(dst3d):
        pltpu.sync_copy(acc_sh.at[pl.ds(r0, RZB)],
                        dst3d.at[cid, pl.ds(r0, RZB)])

        @pl.when(sid == NS - 1)
        def _():
            pltpu.sync_copy(acc_sh.at[pl.ds(t0, RTAIL)],
                            dst3d.at[cid, pl.ds(t0, RTAIL)])

    def idx_copy_async(t, p):
        e0 = e_base + t * CH
        pltpu.async_copy(src_hbm.at[pl.ds(e0, CH)], srcv[p], sem_i[p])
        return pltpu.async_copy(dst_hbm.at[pl.ds(e0, CH)], dstv[p], sem_i[p])

    def gathers_async(t, p):
        e0 = e_base + t * CH
        pltpu.async_copy(ps_hbm.at[srcv[p]], rowsa[p], sem_a[p])
        pltpu.async_copy(pd_hbm.at[dstv[p]], rowsb[p], sem_b[p])
        return pltpu.async_copy(ez_hbm.at[pl.ds(e0, CH)], rowse[p], sem_e[p])

    zero_acc()
    plsc.subcore_barrier()

    idx_copy_async(0, 0)
    pltpu.make_async_copy(src_hbm.at[pl.ds(0, CH)], srcv[0], sem_i[0]).wait()
    pltpu.make_async_copy(dst_hbm.at[pl.ds(0, CH)], dstv[0], sem_i[0]).wait()
    idx_copy_async(1, 1)
    gathers_async(0, 0)

    def chunk(i, carry):
        for b in (0, 1):
            t = 2 * i + b

            pltpu.make_async_copy(ez_hbm.at[pl.ds(0, CH)], rowsa[b],
                                  sem_a[b]).wait()
            pltpu.make_async_copy(ez_hbm.at[pl.ds(0, CH)], rowsb[b],
                                  sem_b[b]).wait()
            pltpu.make_async_copy(ez_hbm.at[pl.ds(0, CH)], rowse[b],
                                  sem_e[b]).wait()

            @pl.when((t >= 1) & (t + 1 < NCHUNK))
            def _():
                pltpu.make_async_copy(rowse[1 - b],
                                      acc_sh.at[dsts[1 - b]],
                                      sem_s[1 - b]).wait()

            @pl.when(t + 1 < NCHUNK)
            def _():
                pltpu.make_async_copy(src_hbm.at[pl.ds(0, CH)],
                                      srcv[1 - b], sem_i[1 - b]).wait()
                pltpu.make_async_copy(dst_hbm.at[pl.ds(0, CH)],
                                      dstv[1 - b], sem_i[1 - b]).wait()
                gathers_async(t + 1, 1 - b)

            def row(rr, c2):
                for cc in range(MEM // 16):
                    sl = pl.ds(cc * 16, 16)
                    v = rowsa[b][rr, sl] + rowsb[b][rr, sl] + rowse[b][rr, sl]
                    rowse[b][rr, sl] = jnp.maximum(v, 0.0)
                return c2

            lax.fori_loop(0, CH, row, 0)
            dsts[b][pl.ds(0, 16)] = dstv[b][pl.ds(0, 16)]
            dsts[b][pl.ds(16, 16)] = dstv[b][pl.ds(16, 16)]
            dsts[b][pl.ds(CH - 16, 16)] = dstv[b][pl.ds(CH - 16, 16)]
            pltpu.async_copy(rowse[b], acc_sh.at[dsts[b]], sem_s[b],
                             add=True)

            @pl.when(t + 2 < NCHUNK)
            def _():
                idx_copy_async(t + 2, b)

        return carry

    lax.fori_loop(0, NCHUNK // 2, chunk, 0)
    pltpu.make_async_copy(rowse[0], acc_sh.at[dsts[0]], sem_s[0]).wait()
    pltpu.make_async_copy(rowse[1], acc_sh.at[dsts[1]], sem_s[1]).wait()
    plsc.subcore_barrier()
    write_acc(out_hbm)
    plsc.subcore_barrier()

    zero_acc()
    pltpu.sync_copy(ones_hbm, rowse[0])
    plsc.subcore_barrier()

    def dst_copy_async(t, p):
        e0 = e_base + t * CH
        return pltpu.async_copy(dst_hbm.at[pl.ds(e0, CH)], dstv[p], sem_i[p])

    dst_copy_async(0, 0).wait()
    dst_copy_async(1, 1)

    def cchunk(i, carry):
        for b in (0, 1):
            t = 2 * i + b

            @pl.when(t >= 2)
            def _():
                pltpu.make_async_copy(rowse[0], acc_sh.at[dsts[b]],
                                      sem_s[b]).wait()

            dsts[b][pl.ds(0, 16)] = dstv[b][pl.ds(0, 16)]
            dsts[b][pl.ds(16, 16)] = dstv[b][pl.ds(16, 16)]
            dsts[b][pl.ds(CH - 16, 16)] = dstv[b][pl.ds(CH - 16, 16)]
            pltpu.async_copy(rowse[0], acc_sh.at[dsts[b]], sem_s[b],
                             add=True)

            @pl.when(t + 2 < NCHUNK)
            def _():
                dst_copy_async(t + 2, b)

            @pl.when(t + 1 < NCHUNK)
            def _():
                pltpu.make_async_copy(dst_hbm.at[pl.ds(0, CH)],
                                      dstv[1 - b], sem_i[1 - b]).wait()

        return carry

    lax.fori_loop(0, NCHUNK // 2, cchunk, 0)
    pltpu.make_async_copy(rowse[0], acc_sh.at[dsts[0]], sem_s[0]).wait()
    pltpu.make_async_copy(rowse[0], acc_sh.at[dsts[1]], sem_s[1]).wait()
    plsc.subcore_barrier()
    write_acc(cnt_hbm)


@functools.lru_cache(maxsize=1)
def _build_edge_agg():
    return functools.partial(
        pl.kernel,
        out_type=[
            jax.ShapeDtypeStruct((NC, N, MEM), _f32),
            jax.ShapeDtypeStruct((NC, N, MEM), _f32),
        ],
        mesh=plsc.VectorSubcoreMesh(
            core_axis_name="c", subcore_axis_name="s",
            num_cores=NC, num_subcores=NS),
        scratch_types=(
            [pltpu.VMEM_SHARED((N, MEM), _f32)]
            + [pltpu.VMEM((CH,), jnp.int32)] * 6
            + [pltpu.VMEM((CH, MEM), _f32)] * 6
            + [pltpu.SemaphoreType.DMA] * 10
        ),
    )(_edge_agg_body)


def kernel(node_x, edge_index, edge_z, W_in, b_in, Wm1, bm1, Wm2, bm2,
           Wmm, bmm, Wz, bz, Wr, br, Wh, bh, Wout, bout):
    src = edge_index[0].astype(jnp.int32)
    dst = edge_index[1].astype(jnp.int32)
    a_w = Wm1[:MEM]
    b_w = Wm1[MEM:2 * MEM]
    c_w = Wm1[2 * MEM:]

    mem, ps, pd = _node_proj(node_x, W_in, b_in.reshape(1, MEM), a_w, b_w)
    ez = _edge_term(edge_z, c_w, bm1.reshape(1, MEM))

    z128 = jnp.zeros((N, MEM), _f32)
    ones = jnp.ones((CH, MEM), _f32)
    partial, cnt = _build_edge_agg()(ps, pd, ez, src, dst, z128, ones)

    emb = _finalize(
        partial[0], partial[1], cnt[0], cnt[1], mem,
        Wm2, bm2.reshape(1, MEM), Wmm, bmm.reshape(1, MEM),
        Wz[:MEM], Wz[MEM:], bz.reshape(1, MEM),
        Wr[:MEM], Wr[MEM:], br.reshape(1, MEM),
        Wh[:MEM], Wh[MEM:], bh.reshape(1, MEM),
        Wout, bout.reshape(1, MEM))
    return emb
